# baseline (speedup 1.0000x reference)
import functools
import jax
import jax.numpy as jnp
from jax import lax
from jax.experimental import pallas as pl
from jax.experimental.pallas import tpu as pltpu, tpu_sc as plsc

N = 10000
E = 320000
F_IN = 128
H = 16
C_OUT = 40
K = 2000

NC, NS = 2, 16
NW = NC * NS
NPAD = 10240
KP = 2048
EP = 327680
EW = EP // NW
CH = 128
NCH = EW // CH
ROWS_T = NPAD // NS
HALF = NPAD // 2
TSTR = 2000
BSH = 9
TSIZE = 512 * TSTR
NB = 20

_mesh = plsc.VectorSubcoreMesh(
    core_axis_name="c", subcore_axis_name="s", num_cores=NC, num_subcores=NS)
_sc_params = pltpu.CompilerParams(use_tc_tiling_on_sc=False,
                                  needs_layout_passes=False)


def _wid():
  cid = lax.axis_index("c")
  sid = lax.axis_index("s")
  return cid, sid, sid * NC + cid


def _iota16():
  return lax.iota(jnp.int32, 16)


def _fill(ref, val):
  if len(ref.shape) == 1:
    for off in range(0, ref.shape[0], 16):
      ref[pl.ds(off, 16)] = jnp.full((16,), val, ref.dtype)
  else:
    assert ref.shape[1] % 16 == 0
    for r in range(ref.shape[0]):
      for cc in range(0, ref.shape[1], 16):
        ref[r, pl.ds(cc, 16)] = jnp.full((16,), val, ref.dtype)

def _zero_spmem(zero_v, sh_ref, start, rows):
  _fill(zero_v, 0.0)
  zb = zero_v.shape[0]
  for off in range(0, rows, zb):
    step = min(zb, rows - off)
    pltpu.sync_copy(zero_v.at[pl.ds(0, step)], sh_ref.at[pl.ds(start + off, step)])


@functools.partial(
    pl.kernel, mesh=_mesh, compiler_params=_sc_params,
    out_type=[jax.ShapeDtypeStruct((NC, NPAD), jnp.float32)],
    scratch_types=[
        pltpu.VMEM((CH,), jnp.int32),
        pltpu.VMEM((CH,), jnp.float32),
        pltpu.VMEM_SHARED((NPAD,), jnp.float32),
        pltpu.VMEM((ROWS_T,), jnp.float32),
    ],
)
def _deg_kernel(dst_hbm, out_hbm, idx_v, ones_v, acc_sh, zero_v):
  cid, sid, wid = _wid()
  _zero_spmem(zero_v, acc_sh, sid * ROWS_T, ROWS_T)
  _fill(ones_v, 1.0)
  plsc.subcore_barrier()

  def body(i, _):
    base = pl.multiple_of(wid * EW + i * CH, CH)
    pltpu.sync_copy(dst_hbm.at[pl.ds(base, CH)], idx_v)
    pltpu.sync_copy(ones_v, acc_sh.at[idx_v], add=True)
    return 0

  lax.fori_loop(0, NCH, body, 0)
  plsc.subcore_barrier()
  pltpu.sync_copy(acc_sh.at[pl.ds(sid * ROWS_T, ROWS_T)],
                  out_hbm.at[cid, pl.ds(sid * ROWS_T, ROWS_T)])


def _make_edge_agg(width):
  @functools.partial(
      pl.kernel, mesh=_mesh, compiler_params=_sc_params,
      out_type=[jax.ShapeDtypeStruct((NC, NPAD, width), jnp.float32)],
      scratch_types=[
          pltpu.VMEM((CH,), jnp.int32),
          pltpu.VMEM((CH,), jnp.int32),
          pltpu.VMEM((CH, width), jnp.float32),
          pltpu.VMEM_SHARED((NPAD, width), jnp.float32),
          pltpu.VMEM((CH, width), jnp.float32),
          pltpu.SemaphoreType.DMA,
      ],
  )
  def _edge_agg(table_hbm, src_hbm, dst_hbm, out_hbm,
                idxa_v, idxb_v, rows_v, acc_sh, zero_v, sem):
    cid, sid, wid = _wid()
    _fill(zero_v, 0.0)
    for q in range(ROWS_T // CH):
      pltpu.sync_copy(zero_v, acc_sh.at[pl.ds(sid * ROWS_T + q * CH, CH)])
    plsc.subcore_barrier()

    def body(i, _):
      base = pl.multiple_of(wid * EW + i * CH, CH)
      pltpu.sync_copy(src_hbm.at[pl.ds(base, CH)], idxa_v)
      pltpu.async_copy(table_hbm.at[idxa_v], rows_v, sem).wait()
      pltpu.sync_copy(dst_hbm.at[pl.ds(base, CH)], idxb_v)
      pltpu.sync_copy(rows_v, acc_sh.at[idxb_v], add=True)
      return 0

    lax.fori_loop(0, NCH, body, 0)
    plsc.subcore_barrier()
    pltpu.sync_copy(acc_sh.at[pl.ds(sid * ROWS_T, ROWS_T)],
                    out_hbm.at[cid, pl.ds(sid * ROWS_T, ROWS_T)])

  return _edge_agg


_edge_agg_npad = _make_edge_agg(H)
_edge_agg_wide = _make_edge_agg(48)


@functools.partial(
    pl.kernel, mesh=_mesh, compiler_params=_sc_params,
    out_type=[jax.ShapeDtypeStruct((KP,), jnp.int32),
              jax.ShapeDtypeStruct((NPAD,), jnp.int32)],
    scratch_types=[
        pltpu.VMEM((ROWS_T,), jnp.int32),
        pltpu.VMEM((48,), jnp.int32),
        pltpu.VMEM((16,), jnp.int32),
        pltpu.VMEM((16,), jnp.int32),
        pltpu.VMEM_SHARED((16,), jnp.int32),
        pltpu.VMEM_SHARED((16,), jnp.int32),
        pltpu.VMEM((ROWS_T,), jnp.int32),
        pltpu.VMEM((5, CH), jnp.int32),
        pltpu.VMEM((5, CH), jnp.int32),
        pltpu.VMEM((48,), jnp.int32),
        pltpu.SemaphoreType.DMA,
    ],
)
def _permsel_kernel(skey_hbm, tg_hbm, perm_hbm, inv_hbm,
                    key_v, tg_v, iota_v, cnt_v, gt_sh, eq_sh,
                    inv_v, rank_v, node_v, pad_v, sem):
  cid, sid, _ = _wid()
  base = sid * ROWS_T
  ii = _iota16()

  @pl.when(cid == 0)
  def _():
    pltpu.sync_copy(skey_hbm.at[pl.ds(base, ROWS_T)], key_v)
    pltpu.sync_copy(tg_hbm, tg_v)
    iota_v[...] = ii

    @pl.when(sid == 0)
    def _():
      cnt_v[...] = jnp.zeros_like(cnt_v)
      pltpu.sync_copy(cnt_v, gt_sh)
      pltpu.sync_copy(cnt_v, eq_sh)

  plsc.subcore_barrier()

  @pl.when(cid == 0)
  def _():
    t_vec = tg_v[pl.ds(0, 16)]
    cgt = jnp.int32(0)
    ceq = jnp.int32(0)
    for j in range(ROWS_T // 16):
      k = key_v[pl.ds(j * 16, 16)]
      cgt = cgt + jnp.sum(jnp.where(k > t_vec, 1, 0).astype(jnp.int32))
      ceq = ceq + jnp.sum(jnp.where(k == t_vec, 1, 0).astype(jnp.int32))
    cnt_v[...] = jnp.where(ii == sid, cgt, 0).astype(jnp.int32)
    pltpu.sync_copy(cnt_v, gt_sh.at[iota_v], add=True)
    cnt_v[...] = jnp.where(ii == sid, ceq, 0).astype(jnp.int32)
    pltpu.sync_copy(cnt_v, eq_sh.at[iota_v], add=True)

  plsc.subcore_barrier()

  @pl.when(cid == 0)
  def _():
    t_vec = tg_v[pl.ds(0, 16)]
    g_vec = tg_v[pl.ds(16, 16)]
    need_vec = tg_v[pl.ds(32, 16)]
    pltpu.sync_copy(gt_sh, cnt_v)
    gts = cnt_v[...]
    pgt = plsc.cumsum(gts) - gts
    gbase = jnp.sum(jnp.where(ii == sid, pgt, 0).astype(jnp.int32))
    pltpu.sync_copy(eq_sh, cnt_v)
    eqs = cnt_v[...]
    peq = plsc.cumsum(eqs) - eqs
    ebase = jnp.sum(jnp.where(ii == sid, peq, 0).astype(jnp.int32))

    g_run = jnp.int32(0)
    e_run = jnp.int32(0)
    for j in range(ROWS_T // 16):
      k = key_v[pl.ds(j * 16, 16)]
      m_gt = k > t_vec
      m_eq = k == t_vec
      gi = jnp.where(m_gt, 1, 0).astype(jnp.int32)
      ei = jnp.where(m_eq, 1, 0).astype(jnp.int32)
      cg = plsc.cumsum(gi)
      ce = plsc.cumsum(ei)
      grank = gbase + g_run + cg - 1
      erank = ebase + e_run + ce - 1
      sel_eq = m_eq & (erank < need_vec)
      rank = jnp.where(m_gt, grank, g_vec + erank)
      sel = m_gt | sel_eq
      node = base + j * 16 + ii
      inv_v[pl.ds(j * 16, 16)] = jnp.where(sel, rank, K).astype(jnp.int32)
      rank_v[j // 8, pl.ds((j % 8) * 16, 16)] = jnp.where(
          sel, rank, 2016 + ii).astype(jnp.int32)
      node_v[j // 8, pl.ds((j % 8) * 16, 16)] = node.astype(jnp.int32)
      g_run = g_run + jnp.sum(gi)
      e_run = e_run + jnp.sum(ei)

    pltpu.sync_copy(inv_v, inv_hbm.at[pl.ds(base, ROWS_T)])
    for c in range(5):
      pltpu.async_copy(node_v.at[c], perm_hbm.at[rank_v.at[c]], sem).wait()

  plsc.subcore_barrier()

  @pl.when((cid == 0) & (sid == 0))
  def _():
    _fill(pad_v, N)
    pltpu.sync_copy(pad_v, perm_hbm.at[pl.ds(K, 48)])


@functools.partial(
    pl.kernel, mesh=_mesh, compiler_params=_sc_params,
    out_type=[jax.ShapeDtypeStruct((KP, H), jnp.float32),
              jax.ShapeDtypeStruct((KP, H), jnp.float32),
              jax.ShapeDtypeStruct((NW, 16), jnp.int32),
              jax.ShapeDtypeStruct((NW, 16), jnp.int32)],
    scratch_types=[
        pltpu.VMEM((NPAD,), jnp.int32),
        pltpu.VMEM((CH,), jnp.int32),
        pltpu.VMEM((CH,), jnp.int32),
        pltpu.VMEM((16,), jnp.int32),
        pltpu.VMEM((16,), jnp.int32),
        pltpu.VMEM((64,), jnp.int32),
        pltpu.VMEM((64, H), jnp.float32),
        pltpu.SemaphoreType.DMA,
    ],
)
def _afirst_kernel(src_hbm, dst_hbm, inv_hbm, x1s_hbm, x1_hbm, perm_hbm,
                   xp_out, x1perm_out, cntU_out, cntV_out,
                   inv_v, idxs_v, idxd_v, cntU_v, cntV_v,
                   pidx_v, rows_v, sem):
  cid, sid, wid = _wid()
  pltpu.sync_copy(inv_hbm, inv_v)
  cntU_v[...] = jnp.zeros_like(cntU_v)
  cntV_v[...] = jnp.zeros_like(cntV_v)

  def body(i, _):
    base = pl.multiple_of(wid * EW + i * CH, CH)
    pltpu.sync_copy(src_hbm.at[pl.ds(base, CH)], idxs_v)
    pltpu.sync_copy(dst_hbm.at[pl.ds(base, CH)], idxd_v)
    for l in range(CH // 16):
      s = idxs_v[pl.ds(l * 16, 16)]
      d = idxd_v[pl.ds(l * 16, 16)]
      invd = plsc.load_gather(inv_v, [d])
      invs = plsc.load_gather(inv_v, [s])
      nonself = s != d
      u_e = nonself & (invd < K)
      v_e = nonself & (invs < K)
      bu = lax.shift_right_logical(invd, 7)
      cnts, lastm = plsc.scan_count(bu, u_e)
      plsc.addupdate_scatter(cntU_v, [bu], cnts, mask=lastm)
      bv = lax.shift_right_logical(invs, 7)
      cntsv, lastv = plsc.scan_count(bv, v_e)
      plsc.addupdate_scatter(cntV_v, [bv], cntsv, mask=lastv)
    return 0

  lax.fori_loop(0, NCH, body, 0)

  pbase = wid * 64
  pltpu.sync_copy(perm_hbm.at[pl.ds(pbase, 64)], pidx_v)
  pltpu.async_copy(x1s_hbm.at[pidx_v], rows_v, sem).wait()
  pltpu.sync_copy(rows_v, xp_out.at[pl.ds(pbase, 64)])
  pltpu.async_copy(x1_hbm.at[pidx_v], rows_v, sem).wait()
  pltpu.sync_copy(rows_v, x1perm_out.at[pl.ds(pbase, 64)])
  pltpu.sync_copy(cntU_v, cntU_out.at[wid])
  pltpu.sync_copy(cntV_v, cntV_out.at[wid])


@functools.partial(
    pl.kernel, mesh=_mesh, compiler_params=_sc_params,
    out_type=[jax.ShapeDtypeStruct((EP + 16,), jnp.int32),
              jax.ShapeDtypeStruct((EP + 16,), jnp.int32)],
    scratch_types=[
        pltpu.VMEM((NPAD,), jnp.int32),
        pltpu.VMEM((CH,), jnp.int32),
        pltpu.VMEM((CH,), jnp.int32),
        pltpu.VMEM((CH,), jnp.int32),
        pltpu.VMEM((CH,), jnp.int32),
        pltpu.VMEM((CH,), jnp.int32),
        pltpu.VMEM((CH,), jnp.int32),
        pltpu.VMEM((16,), jnp.int32),
        pltpu.VMEM((16,), jnp.int32),
        pltpu.SemaphoreType.DMA,
    ],
)
def _bucket_scatter_kernel(src_hbm, dst_hbm, inv_hbm, ou_hbm, ov_hbm,
                           uk_out, vk_out,
                           inv_v, idxs_v, idxd_v, ku_v, pu_v, kv_v, pv_v,
                           ou_v, ov_v, sem):
  cid, sid, wid = _wid()
  pltpu.sync_copy(inv_hbm, inv_v)
  pltpu.sync_copy(ou_hbm.at[wid], ou_v)
  pltpu.sync_copy(ov_hbm.at[wid], ov_v)
  ii = _iota16()

  def body(i, _):
    base = pl.multiple_of(wid * EW + i * CH, CH)
    pltpu.sync_copy(src_hbm.at[pl.ds(base, CH)], idxs_v)
    pltpu.sync_copy(dst_hbm.at[pl.ds(base, CH)], idxd_v)
    for l in range(CH // 16):
      s = idxs_v[pl.ds(l * 16, 16)]
      d = idxd_v[pl.ds(l * 16, 16)]
      invd = plsc.load_gather(inv_v, [d])
      invs = plsc.load_gather(inv_v, [s])
      nonself = s != d
      u_e = nonself & (invd < K)
      v_e = nonself & (invs < K)
      bu = lax.shift_right_logical(invd, 7)
      cnts, lastm = plsc.scan_count(bu, u_e)
      obase = plsc.load_gather(ou_v, [bu])
      pos = obase + cnts - 1
      pu_v[pl.ds(l * 16, 16)] = jnp.where(u_e, pos, EP + ii).astype(jnp.int32)
      ku_v[pl.ds(l * 16, 16)] = s * 2048 + invd
      plsc.addupdate_scatter(ou_v, [bu], cnts, mask=lastm)
      bv = lax.shift_right_logical(invs, 7)
      cntsv, lastv = plsc.scan_count(bv, v_e)
      obasev = plsc.load_gather(ov_v, [bv])
      posv = obasev + cntsv - 1
      pv_v[pl.ds(l * 16, 16)] = jnp.where(v_e, posv, EP + ii).astype(jnp.int32)
      kv_v[pl.ds(l * 16, 16)] = d * 2048 + invs
      plsc.addupdate_scatter(ov_v, [bv], cntsv, mask=lastv)
    pltpu.async_copy(ku_v, uk_out.at[pu_v], sem).wait()
    pltpu.async_copy(kv_v, vk_out.at[pv_v], sem).wait()
    return 0

  lax.fori_loop(0, NCH, body, 0)


RCH = 128
TSZ = RCH * NPAD

@functools.partial(
    pl.kernel, mesh=_mesh, compiler_params=_sc_params,
    out_type=[jax.ShapeDtypeStruct((KP * NPAD,), jnp.float32),
              jax.ShapeDtypeStruct((KP * NPAD,), jnp.float32)],
    scratch_types=[
        pltpu.VMEM((CH,), jnp.int32),
        pltpu.VMEM((CH,), jnp.int32),
        pltpu.VMEM((CH,), jnp.float32),
        pltpu.VMEM((16,), jnp.int32),
        pltpu.VMEM((16,), jnp.int32),
        pltpu.VMEM((16,), jnp.int32),
        pltpu.VMEM((16,), jnp.int32),
        pltpu.VMEM((16,), jnp.int32),
        pltpu.VMEM((16,), jnp.float32),
        pltpu.VMEM_SHARED((TSZ + 16,), jnp.float32),
        pltpu.VMEM((1024,), jnp.float32),
        pltpu.SemaphoreType.DMA,
    ],
)
def _bmcm_kernel(uk_hbm, vk_hbm, stu_hbm, enu_hbm, stv_hbm, env_hbm, perm_hbm,
                 ut_out, cm_out,
                 key_v, tidx_v, tval_v, stu_v, enu_v, stv_v, env_v,
                 lidx_v, lval_v, t_sh, zero_v, sem):
  cid, sid, wid = _wid()
  _zero_spmem(zero_v, t_sh, sid * (TSZ // NS), TSZ // NS)
  pltpu.sync_copy(stu_hbm, stu_v)
  pltpu.sync_copy(enu_hbm, enu_v)
  pltpu.sync_copy(stv_hbm, stv_v)
  pltpu.sync_copy(env_hbm, env_v)
  plsc.subcore_barrier()
  ii = _iota16()

  def list_pass(list_hbm, st, en, r0, sign):
    a0 = lax.shift_left(lax.shift_right_logical(st, 7), 7)
    nch = lax.shift_right_logical(en - a0 + 127, 7)
    cnt = jnp.maximum((nch - sid + NS - 1) // NS, 0)
    st_vec = jnp.full((16,), st, jnp.int32)
    en_vec = jnp.full((16,), en, jnp.int32)
    r0_vec = jnp.full((16,), r0, jnp.int32)

    def chunk(j, _):
      base = pl.multiple_of(a0 + (sid + j * NS) * CH, CH)
      pltpu.sync_copy(list_hbm.at[pl.ds(base, CH)], key_v)
      for l in range(CH // 16):
        k = key_v[pl.ds(l * 16, 16)]
        p = base + l * 16 + ii
        mask = (p >= st_vec) & (p < en_vec)
        m = lax.shift_right_logical(k, 11)
        r = (k & 2047) - r0_vec
        tix = r * NPAD + m
        tidx_v[pl.ds(l * 16, 16)] = jnp.where(mask, tix, TSZ).astype(jnp.int32)
        tval_v[pl.ds(l * 16, 16)] = jnp.where(mask, sign, 0.0)
      pltpu.sync_copy(tval_v, t_sh.at[tidx_v], add=True)
      return 0

    lax.fori_loop(0, cnt, chunk, 0)

  def halfloop(list_hbm, st_v_ref, en_v_ref, out_hbm):
    def rchunk(ci, _):
      r0 = (cid * 8 + ci) * RCH
      b = cid * 8 + ci
      st = jnp.sum(jnp.where(ii == b, st_v_ref[...], 0).astype(jnp.int32))
      en = jnp.sum(jnp.where(ii == b, en_v_ref[...], 0).astype(jnp.int32))
      list_pass(list_hbm, st, en, r0, 1.0)
      rr = r0 + sid * 8 + lax.iota(jnp.int32, 16) % 8
      pltpu.sync_copy(perm_hbm.at[pl.ds(pl.multiple_of(r0 + sid * 8, 8), 8)],
                      lidx_v.at[pl.ds(0, 8)])
      pj = lidx_v[pl.ds(0, 16)]
      low = ii < 8
      valid = low & (rr < K)
      lix = (sid * 8 + (ii % 8)) * NPAD + pj
      lidx_v[...] = jnp.where(valid, lix, TSZ).astype(jnp.int32)
      lval_v[...] = jnp.where(valid, 1.0, 0.0)
      pltpu.sync_copy(lval_v, t_sh.at[lidx_v], add=True)
      plsc.subcore_barrier()
      pltpu.sync_copy(
          t_sh.at[pl.ds(sid * (TSZ // NS), TSZ // NS)],
          out_hbm.at[pl.ds((r0 + sid * 8) * NPAD, TSZ // NS)])
      plsc.subcore_barrier()
      list_pass(list_hbm, st, en, r0, -1.0)
      lval_v[...] = jnp.where(valid, -1.0, 0.0)
      pltpu.sync_copy(lval_v, t_sh.at[lidx_v], add=True)
      plsc.subcore_barrier()
      return 0

    lax.fori_loop(0, KP // RCH // NC, rchunk, 0)

  halfloop(uk_hbm, stu_v, enu_v, ut_out)
  halfloop(vk_hbm, stv_v, env_v, cm_out)


@functools.partial(
    pl.kernel, mesh=_mesh, compiler_params=_sc_params,
    out_type=[jax.ShapeDtypeStruct((NC, NPAD, H), jnp.float32)],
    scratch_types=[
        pltpu.VMEM((NPAD,), jnp.int32),
        pltpu.VMEM((CH,), jnp.int32),
        pltpu.VMEM((CH,), jnp.int32),
        pltpu.VMEM((CH,), jnp.int32),
        pltpu.VMEM((CH, H), jnp.float32),
        pltpu.VMEM_SHARED((NPAD, H), jnp.float32),
        pltpu.VMEM((CH, H), jnp.float32),
        pltpu.SemaphoreType.DMA,
    ],
)
def _upass_kernel(src_hbm, dst_hbm, inv_hbm, z_hbm, out_hbm,
                  inv_v, idxs_v, idxd_v, zi_v, rows_v, u_sh, zero_v, sem):
  cid, sid, wid = _wid()
  pltpu.sync_copy(inv_hbm, inv_v)
  _fill(zero_v, 0.0)
  for q in range(ROWS_T // CH):
    pltpu.sync_copy(zero_v, u_sh.at[pl.ds(sid * ROWS_T + q * CH, CH)])
  plsc.subcore_barrier()
  ii = _iota16()

  def body(i, _):
    base = pl.multiple_of(wid * EW + i * CH, CH)
    pltpu.sync_copy(src_hbm.at[pl.ds(base, CH)], idxs_v)
    pltpu.sync_copy(dst_hbm.at[pl.ds(base, CH)], idxd_v)
    for l in range(CH // 16):
      s = idxs_v[pl.ds(l * 16, 16)]
      d = idxd_v[pl.ds(l * 16, 16)]
      invd = plsc.load_gather(inv_v, [d])
      u_e = (s != d) & (invd < K)
      zi_v[pl.ds(l * 16, 16)] = jnp.where(u_e, invd, K + ii).astype(jnp.int32)
    pltpu.async_copy(z_hbm.at[zi_v], rows_v, sem).wait()
    pltpu.sync_copy(rows_v, u_sh.at[idxs_v], add=True)
    return 0

  lax.fori_loop(0, NCH, body, 0)
  plsc.subcore_barrier()
  pltpu.sync_copy(u_sh.at[pl.ds(sid * ROWS_T, ROWS_T)],
                  out_hbm.at[cid, pl.ds(sid * ROWS_T, ROWS_T)])


@functools.partial(
    pl.kernel, mesh=_mesh, compiler_params=_sc_params,
    out_type=[jax.ShapeDtypeStruct((NC, KP, H), jnp.float32)],
    scratch_types=[
        pltpu.VMEM((NPAD,), jnp.int32),
        pltpu.VMEM((CH,), jnp.int32),
        pltpu.VMEM((CH,), jnp.int32),
        pltpu.VMEM((CH,), jnp.int32),
        pltpu.VMEM((CH,), jnp.int32),
        pltpu.VMEM((CH, H), jnp.float32),
        pltpu.VMEM((CH, H), jnp.float32),
        pltpu.VMEM((CH, H), jnp.float32),
        pltpu.VMEM_SHARED((KP, H), jnp.float32),
        pltpu.VMEM((KP // NS, H), jnp.float32),
        pltpu.SemaphoreType.DMA,
    ],
)
def _pass2_kernel(src_hbm, dst_hbm, inv_hbm, z_hbm, u0_hbm, u1_hbm, out_hbm,
                  inv_v, idxs_v, idxd_v, zi_v, mi_v,
                  rows0_v, rows1_v, rowsz_v, mv_sh, zero_v, sem):
  cid, sid, wid = _wid()
  pltpu.sync_copy(inv_hbm, inv_v)
  _fill(zero_v, 0.0)
  pltpu.sync_copy(zero_v, mv_sh.at[pl.ds(sid * (KP // NS), KP // NS)])
  plsc.subcore_barrier()
  ii = _iota16()

  def body(i, _):
    base = pl.multiple_of(wid * EW + i * CH, CH)
    pltpu.sync_copy(src_hbm.at[pl.ds(base, CH)], idxs_v)
    pltpu.sync_copy(dst_hbm.at[pl.ds(base, CH)], idxd_v)
    for l in range(CH // 16):
      s = idxs_v[pl.ds(l * 16, 16)]
      d = idxd_v[pl.ds(l * 16, 16)]
      invd = plsc.load_gather(inv_v, [d])
      invs = plsc.load_gather(inv_v, [s])
      nonself = s != d
      v_e = nonself & (invs < K)
      s_e = v_e & (invd < K)
      zi_v[pl.ds(l * 16, 16)] = jnp.where(s_e, invd, K + ii).astype(jnp.int32)
      mi_v[pl.ds(l * 16, 16)] = jnp.where(v_e, invs, K + ii).astype(jnp.int32)
    pltpu.async_copy(u0_hbm.at[idxd_v], rows0_v, sem).wait()
    pltpu.async_copy(u1_hbm.at[idxd_v], rows1_v, sem).wait()
    pltpu.async_copy(z_hbm.at[zi_v], rowsz_v, sem).wait()

    def addrows(r, _):
      a = rows0_v[r, pl.ds(0, 16)]
      b = rows1_v[r, pl.ds(0, 16)]
      c = rowsz_v[r, pl.ds(0, 16)]
      rows0_v[r, pl.ds(0, 16)] = a + b + 2.0 * c
      return 0

    lax.fori_loop(0, CH, addrows, 0)
    pltpu.sync_copy(rows0_v, mv_sh.at[mi_v], add=True)
    return 0

  lax.fori_loop(0, NCH, body, 0)
  plsc.subcore_barrier()
  pltpu.sync_copy(mv_sh.at[pl.ds(sid * (KP // NS), KP // NS)],
                  out_hbm.at[cid, pl.ds(sid * (KP // NS), KP // NS)])


@functools.partial(
    pl.kernel, mesh=_mesh, compiler_params=_sc_params,
    out_type=[jax.ShapeDtypeStruct((NPAD + 64, H), jnp.float32)],
    scratch_types=[
        pltpu.VMEM((HALF // NS, H), jnp.float32),
        pltpu.VMEM((128,), jnp.int32),
        pltpu.VMEM((128,), jnp.int32),
        pltpu.VMEM((128, H), jnp.float32),
        pltpu.SemaphoreType.DMA,
    ],
)
def _xrs_kernel(base_hbm, perm_hbm, ws_hbm, out_hbm,
                buf_v, pidx_v, tidx_v, rows_v, sem):
  cid, sid, wid = _wid()
  r0 = cid * HALF + sid * (HALF // NS)
  pltpu.sync_copy(base_hbm.at[pl.ds(r0, HALF // NS)], buf_v)
  pltpu.sync_copy(buf_v, out_hbm.at[pl.ds(r0, HALF // NS)])
  plsc.subcore_barrier()
  ii = _iota16()
  pbase = sid * 128
  pltpu.sync_copy(perm_hbm.at[pl.ds(pbase, 128)], pidx_v)
  lo = cid * HALF
  hi = (cid + 1) * HALF
  for l in range(8):
    p = pidx_v[pl.ds(l * 16, 16)]
    own = (p >= lo) & (p < hi)
    tidx_v[pl.ds(l * 16, 16)] = jnp.where(own, p, NPAD + ii).astype(jnp.int32)
  pltpu.sync_copy(ws_hbm.at[pl.ds(pbase, 128)], rows_v)
  pltpu.async_copy(rows_v, out_hbm.at[tidx_v], sem).wait()


def _tc_call(body, out_shapes, *args):
  return pl.pallas_call(
      body,
      out_shape=[jax.ShapeDtypeStruct(s, d) for (s, d) in out_shapes],
      compiler_params=pltpu.CompilerParams(vmem_limit_bytes=100 * 1024 * 1024),
  )(*args)


def _rowmask(shape2d, limit):
  r = lax.broadcasted_iota(jnp.int32, shape2d, 0)
  return r < limit


def _tcB(degp_ref, x_ref, w1_ref, dis_ref, xws_ref):
  deg = degp_ref[0, :] + degp_ref[1, :] + 2.0
  dis = lax.rsqrt(deg).reshape(NPAD, 1)
  dis = jnp.where(_rowmask((NPAD, 1), N), dis, 0.0)
  xw = jnp.dot(x_ref[...], w1_ref[...], preferred_element_type=jnp.float32)
  dis_ref[...] = dis.reshape(NPAD)
  xws_ref[...] = dis * xw


def _tcD(accp_ref, xws_ref, dis_ref, b1_ref, pw_ref,
         x1_ref, x1s_ref, skey_ref, tg_ref):
  acc = accp_ref[0] + accp_ref[1]
  dis = dis_ref[...].reshape(NPAD, 1)
  x1 = jnp.maximum(dis * acc + 2.0 * dis * xws_ref[...] + b1_ref[...][None, :],
                   0.0)
  x1 = jnp.where(_rowmask((NPAD, H), N), x1, 0.0)
  pw = pw_ref[...]
  nrm = jnp.sqrt(jnp.sum(pw * pw))
  sc = jnp.tanh(jnp.dot(x1, pw.reshape(H, 1),
                        preferred_element_type=jnp.float32) / nrm)
  sc = jnp.where(sc == 0.0, 0.0, sc)
  kb = lax.bitcast_convert_type(sc, jnp.int32)
  key = kb ^ (lax.shift_right_arithmetic(kb, 31) & jnp.int32(0x7FFFFFFF))
  ukey = lax.bitcast_convert_type(key, jnp.uint32) + jnp.uint32(0x80000000)
  ukey = jnp.where(_rowmask((NPAD, 1), N), ukey, jnp.uint32(0))

  def bit(i, cand):
    b = 31 - i
    c2 = cand | (jnp.uint32(1) << b)
    cnt = jnp.sum(jnp.where(ukey >= c2, 1, 0).astype(jnp.int32))
    return jnp.where(cnt >= K, c2, cand)

  t = lax.fori_loop(0, 32, bit, jnp.uint32(0))
  g = jnp.sum(jnp.where(ukey > t, 1, 0).astype(jnp.int32))
  t_s = lax.bitcast_convert_type(t ^ jnp.uint32(0x80000000), jnp.int32)
  x1_ref[...] = x1
  x1s_ref[...] = x1 * sc
  keym = jnp.where(_rowmask((NPAD, 1), N), key, jnp.int32(-0x80000000))
  skey_ref[...] = keym.reshape(NPAD)
  tg = jnp.concatenate([
      jnp.full((16,), t_s, jnp.int32),
      jnp.full((16,), g, jnp.int32),
      jnp.full((16,), K - g, jnp.int32)])
  tg_ref[...] = tg


def _tcG(cntU_ref, cntV_ref, xp_ref, w2_ref,
         ou_ref, ov_ref, stu_ref, enu_ref, stv_ref, env_ref, y_ref):
  tri16 = (lax.broadcasted_iota(jnp.int32, (16, 16), 0) >
           lax.broadcasted_iota(jnp.int32, (16, 16), 1)).astype(jnp.float32)
  triW = (lax.broadcasted_iota(jnp.int32, (NW, NW), 0) >
          lax.broadcasted_iota(jnp.int32, (NW, NW), 1)).astype(jnp.float32)
  for cnt_ref, o_ref, st_ref, en_ref in ((cntU_ref, ou_ref, stu_ref, enu_ref),
                                         (cntV_ref, ov_ref, stv_ref, env_ref)):
    cnt = cnt_ref[...].astype(jnp.float32)
    tot = jnp.sum(cnt, axis=0)
    start = jnp.dot(tot.reshape(1, 16), tri16.T,
                    precision=lax.Precision.HIGHEST,
                    preferred_element_type=jnp.float32).reshape(16)
    pref = jnp.dot(triW, cnt, precision=lax.Precision.HIGHEST,
                   preferred_element_type=jnp.float32)
    o_ref[...] = (start[None, :] + pref).astype(jnp.int32)
    st_ref[...] = start.astype(jnp.int32)
    en_ref[...] = (start + tot).astype(jnp.int32)
  y_ref[...] = jnp.dot(xp_ref[...], w2_ref[...],
                       preferred_element_type=jnp.float32)


def _mm_body(cm_ref, ut_ref, a2_ref):
  @pl.when(pl.program_id(2) == 0)
  def _():
    a2_ref[...] = jnp.zeros_like(a2_ref)
  a2_ref[...] += jnp.dot(cm_ref[...], ut_ref[...].T,
                         preferred_element_type=jnp.float32)


def _a2_matmul(cm, ut):
  blk_i, blk_j, blk_k = 512, 512, 2048
  return pl.pallas_call(
      _mm_body,
      grid=(KP // blk_i, KP // blk_j, NPAD // blk_k),
      in_specs=[pl.BlockSpec((blk_i, blk_k), lambda i, j, k: (i, k)),
                pl.BlockSpec((blk_j, blk_k), lambda i, j, k: (j, k))],
      out_specs=pl.BlockSpec((blk_i, blk_j), lambda i, j, k: (i, j)),
      out_shape=jax.ShapeDtypeStruct((KP, KP), jnp.float32),
      compiler_params=pltpu.CompilerParams(
          dimension_semantics=("parallel", "parallel", "arbitrary"),
          vmem_limit_bytes=100 * 1024 * 1024),
  )(cm, ut)


def _tcDeg(a2_ref, dis2_ref):
  i = pl.program_id(0)
  blk = a2_ref.shape[0]
  a2 = a2_ref[...]
  rows = i * blk + lax.broadcasted_iota(jnp.int32, (blk, KP), 0)
  cols = lax.broadcasted_iota(jnp.int32, (blk, KP), 1)
  eye = (rows == cols).astype(jnp.float32)
  adjh = a2 * (1.0 - eye) + 2.0 * eye
  deg2 = jnp.sum(adjh, axis=1)
  dis2_ref[...] = jnp.where(deg2 > 0, 1.0 / jnp.sqrt(deg2), 0.0)


def _deg_call(a2):
  blk = 256
  return pl.pallas_call(
      _tcDeg,
      grid=(KP // blk,),
      in_specs=[pl.BlockSpec((blk, KP), lambda i: (i, 0))],
      out_specs=pl.BlockSpec((blk,), lambda i: (i,)),
      out_shape=jax.ShapeDtypeStruct((KP,), jnp.float32),
      compiler_params=pltpu.CompilerParams(vmem_limit_bytes=100 * 1024 * 1024),
  )(a2)


def _tcP(a2_ref, dis2_ref, dis2b_ref, y_ref, b2_ref, x1perm_ref, w_ref):
  i = pl.program_id(0)
  blk = a2_ref.shape[0]
  a2 = a2_ref[...]
  rows = i * blk + lax.broadcasted_iota(jnp.int32, (blk, KP), 0)
  cols = lax.broadcasted_iota(jnp.int32, (blk, KP), 1)
  eye = (rows == cols).astype(jnp.float32)
  adjh = a2 * (1.0 - eye) + 2.0 * eye
  dis2 = dis2_ref[...]
  dis2_blk = dis2b_ref[...]
  nadj = (dis2_blk[:, None] * adjh) * dis2[None, :]
  p = jnp.dot(nadj, y_ref[...], preferred_element_type=jnp.float32)
  x2 = jnp.maximum(p + b2_ref[...][None, :], 0.0)
  w_ref[...] = x2 + x1perm_ref[...]


def _p_call(a2, dis2, y, b2, x1perm):
  blk = 256
  return pl.pallas_call(
      _tcP,
      grid=(KP // blk,),
      in_specs=[pl.BlockSpec((blk, KP), lambda i: (i, 0)),
                pl.BlockSpec((KP,), lambda i: (0,)),
                pl.BlockSpec((blk,), lambda i: (i,)),
                pl.BlockSpec((KP, H), lambda i: (0, 0)),
                pl.BlockSpec((H,), lambda i: (0,)),
                pl.BlockSpec((blk, H), lambda i: (i, 0))],
      out_specs=pl.BlockSpec((blk, H), lambda i: (i, 0)),
      out_shape=jax.ShapeDtypeStruct((KP, H), jnp.float32),
      compiler_params=pltpu.CompilerParams(vmem_limit_bytes=100 * 1024 * 1024),
  )(a2, dis2, dis2, y, b2, x1perm)


def _tcR(xr_ref, dis_ref, w3_ref, xw3s_ref):
  xr = xr_ref[pl.ds(0, NPAD), :]
  xw3 = jnp.dot(xr, w3_ref[...], preferred_element_type=jnp.float32)
  dis = dis_ref[...].reshape(NPAD, 1)
  pad = jnp.zeros((NPAD, 8), jnp.float32)
  xw3s_ref[...] = jnp.concatenate([dis * xw3, pad], axis=1)


def _tcQ(aggp_ref, xw3s_ref, dis_ref, b3_ref, out_ref):
  agg = aggp_ref[0][:, :C_OUT] + aggp_ref[1][:, :C_OUT]
  dis = dis_ref[...].reshape(NPAD, 1)
  xw3s = xw3s_ref[pl.ds(0, NPAD), pl.ds(0, C_OUT)]
  out = dis * agg + 2.0 * dis * xw3s
  out_ref[...] = out[:N, :] + b3_ref[...][None, :]


@jax.jit
def kernel(x, edge_index, W1, b1, pool_w, W2, b2, W3, b3):
  src = edge_index[0]
  dst = edge_index[1]
  npad_extra = EP - E
  pad_ids = (N + (jnp.arange(npad_extra, dtype=jnp.int32) % 240)).astype(jnp.int32)
  srcp = jnp.concatenate([src, pad_ids])
  dstp = jnp.concatenate([dst, pad_ids])
  xpad = jnp.zeros((NPAD, F_IN), jnp.float32).at[:N].set(x)

  degp = _deg_kernel(dstp)[0]

  dis, xws = _tc_call(_tcB, [((NPAD,), jnp.float32), ((NPAD, H), jnp.float32)],
                      degp, xpad, W1)

  accp = _edge_agg_npad(xws, srcp, dstp)[0]

  x1, x1s, skey, tg = _tc_call(
      _tcD, [((NPAD, H), jnp.float32), ((NPAD, H), jnp.float32),
             ((NPAD,), jnp.int32), ((48,), jnp.int32)],
      accp, xws, dis, b1, pool_w)

  perm, inv = _permsel_kernel(skey, tg)

  xp, x1perm, cntU, cntV = _afirst_kernel(srcp, dstp, inv, x1s, x1, perm)

  ou, ov, stu, enu, stv, env, y = _tc_call(
      _tcG, [((NW, 16), jnp.int32), ((NW, 16), jnp.int32),
             ((16,), jnp.int32), ((16,), jnp.int32),
             ((16,), jnp.int32), ((16,), jnp.int32),
             ((KP, H), jnp.float32)],
      cntU, cntV, xp, W2)

  uk, vk = _bucket_scatter_kernel(srcp, dstp, inv, ou, ov)

  utf, cmf = _bmcm_kernel(uk, vk, stu, enu, stv, env, perm)
  ut = utf.reshape(KP, NPAD)
  cm = cmf.reshape(KP, NPAD)

  a2 = _a2_matmul(cm, ut)

  dis2 = _deg_call(a2)
  w = _p_call(a2, dis2, y, b2, x1perm)

  xr = _xrs_kernel(x1, perm, w)[0]

  xw3s = _tc_call(_tcR, [((NPAD, 48), jnp.float32)], xr, dis, W3)[0]

  aggp = _edge_agg_wide(xw3s, srcp, dstp)[0]

  out = _tc_call(_tcQ, [((N, C_OUT), jnp.float32)],
                 aggp, xw3s, dis, b3)[0]
  return out
